# baseline (speedup 1.0000x reference)
import functools

import jax
import jax.numpy as jnp
from jax import lax
from jax.experimental import pallas as pl
from jax.experimental.pallas import tpu as pltpu
from jax.experimental.pallas import tpu_sc as plsc

NC = 2
NS = 16
L = 16
NW = NC * NS
BATCH = 16384
NF = 32
BPW = BATCH // NW
CH = 4
CL = BPW // CH


def _mf_body(uidx_hbm, iidx_hbm, uf_hbm, if_hbm, ub_hbm, ib_hbm, gb_hbm,
             out_hbm,
             uidx_v, iidx_v, urows_v, irows_v, ub_v, ib_v, gb_v, out_v,
             *sems):
    wid = lax.axis_index("s") * NC + lax.axis_index("c")
    base = wid * BPW

    pltpu.sync_copy(uidx_hbm.at[pl.ds(base, BPW)], uidx_v)
    pltpu.sync_copy(iidx_hbm.at[pl.ds(base, BPW)], iidx_v)

    copies = []
    for c in range(CH):
        o = c * CL
        copies.append(pltpu.async_copy(
            uf_hbm.at[uidx_v.at[pl.ds(o, CL)]],
            urows_v.at[pl.ds(o, CL)], sems[c]))
        copies.append(pltpu.async_copy(
            if_hbm.at[iidx_v.at[pl.ds(o, CL)]],
            irows_v.at[pl.ds(o, CL)], sems[CH + c]))
        copies.append(pltpu.async_copy(
            ub_hbm.at[uidx_v.at[pl.ds(o, CL)]],
            ub_v.at[pl.ds(o, CL)], sems[2 * CH + c]))
        copies.append(pltpu.async_copy(
            ib_hbm.at[iidx_v.at[pl.ds(o, CL)]],
            ib_v.at[pl.ds(o, CL)], sems[3 * CH + c]))
    pltpu.sync_copy(gb_hbm, gb_v)
    for cp in copies:
        cp.wait()

    gbv = gb_v[...]
    lidx = lax.iota(jnp.int32, L)

    def blk_body(blk, carry):
        o = blk * L
        acc = ub_v[pl.ds(o, L)] + ib_v[pl.ds(o, L)] + gbv
        for j in range(L):
            e = o + j
            u0 = urows_v[e, pl.ds(0, L)]
            i0 = irows_v[e, pl.ds(0, L)]
            u1 = urows_v[e, pl.ds(L, L)]
            i1 = irows_v[e, pl.ds(L, L)]
            p = u0 * i0 + u1 * i1
            acc = jnp.where(lidx == j, acc + jnp.sum(p), acc)
        out_v[pl.ds(o, L)] = acc
        return carry

    lax.fori_loop(0, BPW // L, blk_body, 0)
    pltpu.sync_copy(out_v, out_hbm.at[pl.ds(base, BPW)])


@functools.partial(jax.jit, donate_argnums=())
def _mf(uidx, iidx, uf, itf, ub, ib, gb16):
    mesh = plsc.VectorSubcoreMesh(
        core_axis_name="c", subcore_axis_name="s",
        num_cores=NC, num_subcores=NS)
    run = pl.kernel(
        _mf_body,
        out_type=jax.ShapeDtypeStruct((BATCH,), jnp.float32),
        mesh=mesh,
        scratch_types=[
            pltpu.VMEM((BPW,), jnp.int32),
            pltpu.VMEM((BPW,), jnp.int32),
            pltpu.VMEM((BPW, NF), jnp.float32),
            pltpu.VMEM((BPW, NF), jnp.float32),
            pltpu.VMEM((BPW,), jnp.float32),
            pltpu.VMEM((BPW,), jnp.float32),
            pltpu.VMEM((L,), jnp.float32),
            pltpu.VMEM((BPW,), jnp.float32),
        ] + [pltpu.SemaphoreType.DMA] * (4 * CH),
        compiler_params=pltpu.CompilerParams(
            needs_layout_passes=False, use_tc_tiling_on_sc=False),
    )
    return run(uidx, iidx, uf, itf, ub, ib, gb16)


def kernel(user_idx, item_idx, user_factors, item_factors, user_bias,
           item_bias, global_bias):
    gb16 = jnp.broadcast_to(global_bias.astype(jnp.float32), (L,))
    return _mf(user_idx.astype(jnp.int32), item_idx.astype(jnp.int32),
               user_factors, item_factors, user_bias.reshape(-1),
               item_bias.reshape(-1), gb16)
